# baseline (speedup 1.0000x reference)
import functools

import jax
import jax.numpy as jnp
from jax import lax
from jax.experimental import pallas as pl
from jax.experimental.pallas import tpu as pltpu
from jax.experimental.pallas import tpu_sc as plsc

_LANES = 16
_ALIGN = 72
_EXT = 80
_GROUP = 32


def _build_sc_kernel(B, W, D, V):
    info = plsc.get_sparse_core_info()
    NC, NS = info.num_cores, info.num_subcores
    NW = NC * NS
    assert B % (NW * _GROUP) == 0
    rows_per_w = B // NW
    n_groups = rows_per_w // _GROUP
    TAIL = W - _ALIGN

    mesh = plsc.VectorSubcoreMesh(core_axis_name="c", subcore_axis_name="s")

    @functools.partial(
        pl.kernel,
        mesh=mesh,
        out_type=jax.ShapeDtypeStruct((B, W, D), jnp.float32),
        scratch_types=[
            pltpu.VMEM((_GROUP * _EXT,), jnp.int32),
            pltpu.VMEM((W * D,), jnp.float32),
            pltpu.VMEM((W, D), jnp.float32),
            pltpu.VMEM((W, D), jnp.float32),
            pltpu.VMEM((8, D), jnp.float32),
            pltpu.SemaphoreType.DMA,
            pltpu.SemaphoreType.DMA,
            pltpu.SemaphoreType.DMA,
            pltpu.SemaphoreType.DMA,
        ],
    )
    def gather_add(idx_hbm, table_hbm, ctx_hbm, out_hbm,
                   idx_v, ctx_v, buf0, buf1, mini,
                   g0, g1, s0, s1):
        bufs, gsems, ssems = (buf0, buf1), (g0, g1), (s0, s1)
        wid = lax.axis_index("s") * NC + lax.axis_index("c")
        base = wid * rows_per_w

        pltpu.sync_copy(ctx_hbm, ctx_v)

        def main_copy(jl, i):
            return pltpu.make_async_copy(
                table_hbm.at[idx_v.at[pl.ds(jl * _EXT, _ALIGN)]],
                bufs[i].at[pl.ds(0, _ALIGN)], gsems[i])

        def mini_copy(jl, i):
            return pltpu.make_async_copy(
                table_hbm.at[idx_v.at[pl.ds(jl * _EXT + _ALIGN, 8)]],
                mini, gsems[i])

        def store(j, i):
            return pltpu.make_async_copy(
                bufs[i], out_hbm.at[base + j], ssems[i])

        def add_all(i):
            buf = bufs[i]

            def body(c, _):
                o = c * _LANES
                sl = pl.ds(o, _LANES)
                for r in range(_ALIGN):
                    buf[r, sl] = buf[r, sl] + ctx_v[pl.ds(r * D + o, _LANES)]
                for t in range(TAIL):
                    r = _ALIGN + t
                    buf[r, sl] = mini[8 - TAIL + t, sl] + ctx_v[
                        pl.ds(r * D + o, _LANES)]
                return 0

            lax.fori_loop(0, D // _LANES, body, 0, unroll=False)

        def group(g, _):
            gb = g * _GROUP
            pltpu.sync_copy(idx_hbm.at[pl.ds((base + gb) * _EXT,
                                             _GROUP * _EXT)], idx_v)
            main_copy(0, 0).start()
            mini_copy(0, 0).start()

            def pair(p, _):
                jl0 = 2 * p
                main_copy(jl0, 0).wait()
                mini_copy(jl0, 0).wait()

                @pl.when(p > 0)
                def _():
                    store(0, 1).wait()
                main_copy(jl0 + 1, 1).start()
                add_all(0)
                mini_copy(jl0 + 1, 1).start()
                store(gb + jl0, 0).start()

                main_copy(jl0 + 1, 1).wait()
                mini_copy(jl0 + 1, 1).wait()
                store(0, 0).wait()

                @pl.when(p < _GROUP // 2 - 1)
                def _():
                    main_copy(jl0 + 2, 0).start()
                add_all(1)

                @pl.when(p < _GROUP // 2 - 1)
                def _():
                    mini_copy(jl0 + 2, 0).start()
                store(gb + jl0 + 1, 1).start()
                return 0

            lax.fori_loop(0, _GROUP // 2, pair, 0, unroll=False)
            store(0, 1).wait()
            return 0

        lax.fori_loop(0, n_groups, group, 0, unroll=False)

    return gather_add


def kernel(prompts, token_embedding, ctx):
    B, W = prompts.shape
    V, D = token_embedding.shape
    p32 = prompts.astype(jnp.int32)
    idx_ext = jnp.concatenate([p32[:, :_ALIGN], p32[:, W - 8:]], axis=1)
    sc = _build_sc_kernel(B, W, D, V)
    return sc(idx_ext.reshape(-1), token_embedding, ctx.reshape(-1))


# speedup vs baseline: 2.3433x; 2.3433x over previous
import functools

import jax
import jax.numpy as jnp
from jax import lax
from jax.experimental import pallas as pl
from jax.experimental.pallas import tpu as pltpu
from jax.experimental.pallas import tpu_sc as plsc

_LANES = 16
_ALIGN = 72
_EXT = 80
_GROUP = 32


def _build_sc_kernel(B, W, D, V):
    info = plsc.get_sparse_core_info()
    NC, NS = info.num_cores, info.num_subcores
    NW = NC * NS
    assert B % (NW * _GROUP) == 0
    rows_per_w = B // NW
    n_groups = rows_per_w // _GROUP
    TAIL = W - _ALIGN

    mesh = plsc.VectorSubcoreMesh(core_axis_name="c", subcore_axis_name="s")

    @functools.partial(
        pl.kernel,
        mesh=mesh,
        out_type=jax.ShapeDtypeStruct((B, W, D), jnp.float32),
        scratch_types=[
            pltpu.VMEM((_GROUP * _EXT,), jnp.int32),
            pltpu.VMEM((W * D,), jnp.float32),
            pltpu.VMEM((W, D), jnp.float32),
            pltpu.VMEM((W, D), jnp.float32),
            pltpu.VMEM((8, D), jnp.float32),
            pltpu.SemaphoreType.DMA,
            pltpu.SemaphoreType.DMA,
            pltpu.SemaphoreType.DMA,
            pltpu.SemaphoreType.DMA,
        ],
    )
    def gather_add(idx_hbm, table_hbm, ctx_hbm, out_hbm,
                   idx_v, ctx_v, buf0, buf1, mini,
                   g0, g1, s0, s1):
        bufs, gsems, ssems = (buf0, buf1), (g0, g1), (s0, s1)
        wid = lax.axis_index("s") * NC + lax.axis_index("c")
        base = wid * rows_per_w

        pltpu.sync_copy(ctx_hbm, ctx_v)

        def main_copy(jl, i):
            return pltpu.make_async_copy(
                table_hbm.at[idx_v.at[pl.ds(jl * _EXT, _ALIGN)]],
                bufs[i].at[pl.ds(0, _ALIGN)], gsems[i])

        def mini_copy(jl, i):
            return pltpu.make_async_copy(
                table_hbm.at[idx_v.at[pl.ds(jl * _EXT + _ALIGN, 8)]],
                mini, gsems[i])

        def store(j, i):
            return pltpu.make_async_copy(
                bufs[i], out_hbm.at[base + j], ssems[i])

        def add_all(i):
            buf = bufs[i]

            @functools.partial(plsc.parallel_loop, 0, D // _LANES,
                               unroll=2)
            def _(c):
                o = c * _LANES
                sl = pl.ds(o, _LANES)
                for r in range(_ALIGN):
                    buf[r, sl] = buf[r, sl] + ctx_v[pl.ds(r * D + o, _LANES)]
                for t in range(TAIL):
                    r = _ALIGN + t
                    buf[r, sl] = mini[8 - TAIL + t, sl] + ctx_v[
                        pl.ds(r * D + o, _LANES)]

        def group(g, _):
            gb = g * _GROUP
            pltpu.sync_copy(idx_hbm.at[pl.ds((base + gb) * _EXT,
                                             _GROUP * _EXT)], idx_v)
            main_copy(0, 0).start()
            mini_copy(0, 0).start()

            def pair(p, _):
                jl0 = 2 * p
                main_copy(jl0, 0).wait()
                mini_copy(jl0, 0).wait()

                @pl.when(p > 0)
                def _():
                    store(0, 1).wait()
                main_copy(jl0 + 1, 1).start()
                add_all(0)
                mini_copy(jl0 + 1, 1).start()
                store(gb + jl0, 0).start()

                main_copy(jl0 + 1, 1).wait()
                mini_copy(jl0 + 1, 1).wait()
                store(0, 0).wait()

                @pl.when(p < _GROUP // 2 - 1)
                def _():
                    main_copy(jl0 + 2, 0).start()
                add_all(1)

                @pl.when(p < _GROUP // 2 - 1)
                def _():
                    mini_copy(jl0 + 2, 0).start()
                store(gb + jl0 + 1, 1).start()
                return 0

            lax.fori_loop(0, _GROUP // 2, pair, 0, unroll=False)
            store(0, 1).wait()
            return 0

        lax.fori_loop(0, n_groups, group, 0, unroll=False)

    return gather_add


def kernel(prompts, token_embedding, ctx):
    B, W = prompts.shape
    V, D = token_embedding.shape
    p32 = prompts.astype(jnp.int32)
    idx_ext = jnp.concatenate([p32[:, :_ALIGN], p32[:, W - 8:]], axis=1)
    sc = _build_sc_kernel(B, W, D, V)
    return sc(idx_ext.reshape(-1), token_embedding, ctx.reshape(-1))
